# Initial kernel scaffold; baseline (speedup 1.0000x reference)
import jax
import jax.numpy as jnp
from jax.experimental import pallas as pl


def kernel(x):
    raise NotImplementedError("write your pallas kernel here")


# speedup vs baseline: 79.4508x; 79.4508x over previous
import functools

import jax
import jax.numpy as jnp
from jax import lax
from jax.experimental import pallas as pl
from jax.experimental.pallas import tpu as pltpu

_B, _T, _D = 4096, 32, 2048
_BR = 8


def _minmax_body(x_ref, mn_ref, mx_ref):
    i = pl.program_id(0)
    bmn = jnp.min(x_ref[...])
    bmx = jnp.max(x_ref[...])

    @pl.when(i == 0)
    def _init():
        mn_ref[0, 0] = bmn
        mx_ref[0, 0] = bmx

    @pl.when(i != 0)
    def _acc():
        mn_ref[0, 0] = jnp.minimum(mn_ref[0, 0], bmn)
        mx_ref[0, 0] = jnp.maximum(mx_ref[0, 0], bmx)


def _spike_body(mn_ref, mx_ref, x_ref, out_ref):
    mn = mn_ref[0, 0]
    mx = mx_ref[0, 0]
    xb = x_ref[...]
    xn = (xb - mn) / (mx - mn + jnp.float32(1e-6))
    t = ((jnp.float32(1.0) - xn) * jnp.float32(_T - 1)).astype(jnp.int32)
    tt = lax.broadcasted_iota(jnp.int32, (_BR, _T, _D), 1)
    out_ref[...] = (tt == t[:, None, :]).astype(jnp.float32)


def kernel(x):
    mn, mx = pl.pallas_call(
        _minmax_body,
        grid=(16,),
        in_specs=[pl.BlockSpec((_B // 16, _D), lambda i: (i, 0))],
        out_specs=[
            pl.BlockSpec((1, 1), lambda i: (0, 0), memory_space=pltpu.SMEM),
            pl.BlockSpec((1, 1), lambda i: (0, 0), memory_space=pltpu.SMEM),
        ],
        out_shape=[
            jax.ShapeDtypeStruct((1, 1), jnp.float32),
            jax.ShapeDtypeStruct((1, 1), jnp.float32),
        ],
    )(x)

    spikes = pl.pallas_call(
        _spike_body,
        grid=(_B // _BR,),
        in_specs=[
            pl.BlockSpec((1, 1), lambda i: (0, 0), memory_space=pltpu.SMEM),
            pl.BlockSpec((1, 1), lambda i: (0, 0), memory_space=pltpu.SMEM),
            pl.BlockSpec((_BR, _D), lambda i: (i, 0)),
        ],
        out_specs=pl.BlockSpec((_BR, _T, _D), lambda i: (i, 0, 0)),
        out_shape=jax.ShapeDtypeStruct((_B, _T, _D), jnp.float32),
    )(mn, mx, x)
    return spikes
